# baseline (speedup 1.0000x reference)
import functools

import jax
import jax.numpy as jnp
from jax import lax
from jax.experimental import pallas as pl
from jax.experimental.pallas import tpu as pltpu
from jax.experimental.pallas import tpu_sc as plsc

N = 16384
K = 256
ROWS, COLS = 360, 180
P = ROWS * COLS
NC, NS, LANES = 2, 16, 16
NW = NC * NS
BPW = N // NW
CHUNK = 64
NCHUNK = BPW // CHUNK

_mesh = plsc.VectorSubcoreMesh(core_axis_name="c", subcore_axis_name="s")


@functools.partial(
    pl.kernel,
    out_type=jax.ShapeDtypeStruct((N, K), jnp.float32),
    mesh=_mesh,
    scratch_types=[
        pltpu.VMEM((BPW,), jnp.float32),
        pltpu.VMEM((BPW,), jnp.float32),
        pltpu.VMEM((3, CHUNK), jnp.int32),
        pltpu.VMEM((3, CHUNK), jnp.int32),
        pltpu.VMEM((3, CHUNK), jnp.float32),
        pltpu.VMEM((3, CHUNK), jnp.float32),
        pltpu.VMEM((3, CHUNK, K), jnp.float32),
        pltpu.VMEM((3, CHUNK, K), jnp.float32),
        pltpu.SemaphoreType.DMA,
        pltpu.SemaphoreType.DMA,
        pltpu.SemaphoreType.DMA,
    ],
)
def _sc_lookup(table, lon_in, lat_in, out, lon_v, lat_v, if_v, ic_v, wf_v,
               wc_v, bf, bc, semf, semc, semo):
    wid = lax.axis_index("s") * NC + lax.axis_index("c")
    base = wid * BPW
    pltpu.sync_copy(lon_in.at[pl.ds(base, BPW)], lon_v)
    pltpu.sync_copy(lat_in.at[pl.ds(base, BPW)], lat_v)

    def stage(ch, buf):
        for s in range(CHUNK // LANES):
            sl = pl.ds(s * LANES, LANES)
            ssl = pl.ds(ch * CHUNK + s * LANES, LANES)
            r = lon_v[ssl] + 180.0
            c = lat_v[ssl] + 90.0
            fr = r.astype(jnp.int32)
            fc = c.astype(jnp.int32)
            fa = r - fr.astype(jnp.float32)
            fb = c - fc.astype(jnp.float32)
            cr = jnp.where(fa > 0.0, fr + 1, fr)
            cc = jnp.where(fb > 0.0, fc + 1, fc)
            frc = jnp.minimum(fr, ROWS - 1)
            fcc = jnp.minimum(fc, COLS - 1)
            crc = jnp.minimum(cr, ROWS - 1)
            ccc = jnp.minimum(cc, COLS - 1)
            if_v[buf, sl] = fcc * ROWS + frc
            ic_v[buf, sl] = ccc * ROWS + crc
            omb = 1.0 - fb
            wf_v[buf, sl] = (1.0 - fa) * omb
            wc_v[buf, sl] = fa * omb
        pltpu.async_copy(table.at[if_v.at[buf]], bf.at[buf], semf)
        pltpu.async_copy(table.at[ic_v.at[buf]], bc.at[buf], semc)

    def drain(sem, dst):
        pltpu.make_async_copy(table.at[pl.ds(0, CHUNK)], dst, sem).wait()

    stage(0, 0)
    stage(1, 1)

    def chunk_body(ch, carry):
        slot = lax.rem(ch, 3)
        nslot = lax.rem(ch + 2, 3)

        @pl.when(ch >= 1)
        def _():
            drain(semo, bc.at[nslot])

        @pl.when(ch + 2 < NCHUNK)
        def _():
            stage(ch + 2, nslot)

        drain(semf, bf.at[slot])
        drain(semc, bc.at[slot])

        def combine(g, carry2):
            gbase = g * LANES
            wf16 = wf_v[slot, pl.ds(gbase, LANES)]
            wc16 = wc_v[slot, pl.ds(gbase, LANES)]
            for l in range(LANES):
                wfp = jnp.full((LANES,), wf16[l], jnp.float32)
                wcp = jnp.full((LANES,), wc16[l], jnp.float32)
                p = gbase + l
                for j in range(K // LANES):
                    js = pl.ds(j * LANES, LANES)
                    bf[slot, p, js] = (wfp * bf[slot, p, js]
                                       + wcp * bc[slot, p, js])
            return carry2
        lax.fori_loop(0, CHUNK // LANES, combine, 0)
        cbase = base + ch * CHUNK
        pltpu.async_copy(bf.at[slot], out.at[pl.ds(cbase, CHUNK)], semo)
        return carry

    lax.fori_loop(0, NCHUNK, chunk_body, 0)
    drain(semo, bc.at[0])


def kernel(lonlat, Ys):
    table = jnp.transpose(Ys, (2, 1, 0)).reshape(P, K)
    return _sc_lookup(table, lonlat[:, 0], lonlat[:, 1])
